# baseline (speedup 1.0000x reference)
import functools
import math

import jax
import jax.numpy as jnp
from jax import lax
from jax.experimental import pallas as pl
from jax.experimental.pallas import tpu as pltpu
from jax.experimental.pallas import tpu_sc as plsc

B = 16384
L = 50
D = 64
LANES = 16
NC, NS = 2, 16
NW = NC * NS
BPW = B // NW
HALF = BPW // 2
DT, DI = D // 8, 8
BT, BI = B // 128, 128
SCALE = math.sqrt(D)


def _build():
  mesh = plsc.VectorSubcoreMesh(core_axis_name="c", subcore_axis_name="s")

  @functools.partial(
      pl.kernel,
      out_type=jax.ShapeDtypeStruct((L, DT, BT, DI, BI), jnp.float32),
      mesh=mesh,
      compiler_params=pltpu.CompilerParams(
          use_tc_tiling_on_sc=False, needs_layout_passes=False),
      scratch_types=[
          pltpu.VMEM((L, BPW), jnp.int32),
          pltpu.VMEM((HALF, D), jnp.float32),
          pltpu.VMEM((HALF, D), jnp.float32),
          pltpu.VMEM((DT, 2, DI, BI), jnp.float32),
          pltpu.VMEM((DT, 2, DI, BI), jnp.float32),
          pltpu.SemaphoreType.DMA,
          pltpu.SemaphoreType.DMA,
      ],
  )
  def embed(xt_hbm, table_hbm, out_hbm, idx_all, rows0, rows1, tile0, tile1,
            gsem0, gsem1):
    wid = lax.axis_index("s") * NC + lax.axis_index("c")
    rows_b = (rows0, rows1)
    tile_b = (tile0, tile1)
    gsem = (gsem0, gsem1)
    iota = lax.iota(jnp.int32, LANES)

    def fire(l, h, bufi):
      for j in range(2):
        idx_sl = idx_all.at[l, pl.ds(HALF * h + 128 * j, 128)]
        pltpu.async_copy(table_hbm.at[idx_sl],
                         rows_b[bufi].at[pl.ds(128 * j, 128)], gsem[bufi])

    def drain(bufi):
      for j in range(2):
        idx_sl = idx_all.at[0, pl.ds(128 * j, 128)]
        pltpu.make_async_copy(table_hbm.at[idx_sl],
                              rows_b[bufi].at[pl.ds(128 * j, 128)],
                              gsem[bufi]).wait()

    def fill_store(l, h, bufi):
      r = rows_b[bufi]
      o = tile_b[bufi]
      for btl in range(2):
        @pl.loop(0, DT)
        def _(dt):
          @pl.loop(0, DI)
          def _(di):
            col = jnp.full((LANES,), 8 * dt + di, jnp.int32)
            for bg in range(8):
              i0 = 128 * btl + 16 * bg + iota
              v = plsc.load_gather(r, [i0, col])
              o[dt, btl, di, pl.ds(16 * bg, 16)] = v * SCALE

      bt0 = 4 * wid + 2 * h
      pltpu.sync_copy(o, out_hbm.at[l, :, pl.ds(bt0, 2)])

    pltpu.sync_copy(xt_hbm.at[:, pl.ds(BPW * wid, BPW)], idx_all)
    fire(0, 0, 0)

    @pl.loop(0, L)
    def _(l):
      fire(l, 1, 1)
      drain(0)
      fill_store(l, 0, 0)

      @pl.when(l + 1 < L)
      def _():
        fire(l + 1, 0, 0)

      drain(1)
      fill_store(l, 1, 1)

  return embed


@jax.jit
def kernel(x, table):
  xt = jnp.swapaxes(x.astype(jnp.int32), 0, 1)
  out5 = _build()(xt, table)
  return out5.transpose(2, 4, 0, 1, 3).reshape(B, L, D)


# speedup vs baseline: 1.2042x; 1.2042x over previous
import functools
import math

import jax
import jax.numpy as jnp
from jax import lax
from jax.experimental import pallas as pl
from jax.experimental.pallas import tpu as pltpu
from jax.experimental.pallas import tpu_sc as plsc

B = 16384
L = 50
D = 64
LANES = 16
NC, NS = 2, 16
NW = NC * NS
BPW = B // NW
HALF = BPW // 2
DT, DI = D // 8, 8
BT, BI = B // 128, 128
SCALE = math.sqrt(D)


def _build():
  mesh = plsc.VectorSubcoreMesh(core_axis_name="c", subcore_axis_name="s")

  @functools.partial(
      pl.kernel,
      out_type=jax.ShapeDtypeStruct((L, DT, BT, DI, BI), jnp.float32),
      mesh=mesh,
      compiler_params=pltpu.CompilerParams(
          use_tc_tiling_on_sc=False, needs_layout_passes=False),
      scratch_types=[
          pltpu.VMEM((L, BPW), jnp.int32),
          pltpu.VMEM((HALF, D), jnp.float32),
          pltpu.VMEM((HALF, D), jnp.float32),
          pltpu.VMEM((HALF, D + 1), jnp.float32),
          pltpu.VMEM((DT, 2, DI, BI), jnp.float32),
          pltpu.VMEM((DT, 2, DI, BI), jnp.float32),
          pltpu.SemaphoreType.DMA,
          pltpu.SemaphoreType.DMA,
      ],
  )
  def embed(xt_hbm, table_hbm, out_hbm, idx_all, rows0, rows1, pad65,
            tile0, tile1, gsem0, gsem1):
    wid = lax.axis_index("s") * NC + lax.axis_index("c")
    rows_b = (rows0, rows1)
    tile_b = (tile0, tile1)
    gsem = (gsem0, gsem1)
    iota = lax.iota(jnp.int32, LANES)

    def fire(l, h, bufi):
      for j in range(2):
        idx_sl = idx_all.at[l, pl.ds(HALF * h + 128 * j, 128)]
        pltpu.async_copy(table_hbm.at[idx_sl],
                         rows_b[bufi].at[pl.ds(128 * j, 128)], gsem[bufi])

    def drain(bufi):
      for j in range(2):
        idx_sl = idx_all.at[0, pl.ds(128 * j, 128)]
        pltpu.make_async_copy(table_hbm.at[idx_sl],
                              rows_b[bufi].at[pl.ds(128 * j, 128)],
                              gsem[bufi]).wait()

    def fill_store(l, h, bufi):
      r = rows_b[bufi]
      o = tile_b[bufi]

      @pl.loop(0, HALF, unroll=4)
      def _(row):
        for c in range(D // LANES):
          sl = pl.ds(16 * c, 16)
          pad65[row, sl] = r[row, sl] * SCALE

      for btl in range(2):
        @pl.loop(0, DT)
        def _(dt):
          @pl.loop(0, DI)
          def _(di):
            col = jnp.full((LANES,), 8 * dt + di, jnp.int32)
            for bg in range(8):
              i0 = 128 * btl + 16 * bg + iota
              v = plsc.load_gather(pad65, [i0, col])
              o[dt, btl, di, pl.ds(16 * bg, 16)] = v

      bt0 = 4 * wid + 2 * h
      pltpu.sync_copy(o, out_hbm.at[l, :, pl.ds(bt0, 2)])

    pltpu.sync_copy(xt_hbm.at[:, pl.ds(BPW * wid, BPW)], idx_all)
    fire(0, 0, 0)

    @pl.loop(0, L)
    def _(l):
      fire(l, 1, 1)
      drain(0)
      fill_store(l, 0, 0)

      @pl.when(l + 1 < L)
      def _():
        fire(l + 1, 0, 0)

      drain(1)
      fill_store(l, 1, 1)

  return embed


@jax.jit
def kernel(x, table):
  xt = jnp.swapaxes(x.astype(jnp.int32), 0, 1)
  out5 = _build()(xt, table)
  return out5.transpose(2, 4, 0, 1, 3).reshape(B, L, D)


# speedup vs baseline: 1.7526x; 1.4555x over previous
import functools
import math

import jax
import jax.numpy as jnp
from jax import lax
from jax.experimental import pallas as pl
from jax.experimental.pallas import tpu as pltpu
from jax.experimental.pallas import tpu_sc as plsc

B = 16384
L = 50
D = 64
LANES = 16
NC, NS = 2, 16
NW = NC * NS
BPW = B // NW
HALF = BPW // 2
DT, DI = D // 8, 8
BT, BI = B // 128, 128
SCALE = math.sqrt(D)


def _build():
  mesh = plsc.VectorSubcoreMesh(core_axis_name="c", subcore_axis_name="s")

  @functools.partial(
      pl.kernel,
      out_type=jax.ShapeDtypeStruct((L, DT, BT, DI, BI), jnp.float32),
      mesh=mesh,
      compiler_params=pltpu.CompilerParams(
          use_tc_tiling_on_sc=False, needs_layout_passes=False),
      scratch_types=[
          pltpu.VMEM((L, BPW), jnp.int32),
          pltpu.VMEM((HALF, D), jnp.float32),
          pltpu.VMEM((HALF, D), jnp.float32),
          pltpu.VMEM((2, DT, DI, BI + 1), jnp.float32),
          pltpu.VMEM((2, DT, DI, BI + 1), jnp.float32),
          pltpu.SemaphoreType.DMA,
          pltpu.SemaphoreType.DMA,
      ],
  )
  def embed(xt_hbm, table_hbm, out_hbm, idx_all, rows0, rows1,
            tile0, tile1, gsem0, gsem1):
    wid = lax.axis_index("s") * NC + lax.axis_index("c")
    rows_b = (rows0, rows1)
    tile_b = (tile0, tile1)
    gsem = (gsem0, gsem1)
    iota = lax.iota(jnp.int32, LANES)

    def fire(l, h, bufi):
      for j in range(2):
        idx_sl = idx_all.at[l, pl.ds(HALF * h + 128 * j, 128)]
        pltpu.async_copy(table_hbm.at[idx_sl],
                         rows_b[bufi].at[pl.ds(128 * j, 128)], gsem[bufi])

    def drain(bufi):
      for j in range(2):
        idx_sl = idx_all.at[0, pl.ds(128 * j, 128)]
        pltpu.make_async_copy(table_hbm.at[idx_sl],
                              rows_b[bufi].at[pl.ds(128 * j, 128)],
                              gsem[bufi]).wait()

    dtv = [(16 * c + iota) >> 3 for c in range(D // LANES)]
    div = iota & 7

    def fill_store(l, h, bufi):
      r = rows_b[bufi]
      o = tile_b[bufi]
      for btl in range(2):
        bv = jnp.full((LANES,), btl, jnp.int32)

        @pl.loop(0, 128, unroll=4)
        def _(rr):
          row = 128 * btl + rr
          biv = jnp.full((LANES,), rr, jnp.int32)
          for c in range(D // LANES):
            v = r[row, pl.ds(16 * c, 16)] * SCALE
            plsc.store_scatter(o, [bv, dtv[c], div, biv], v)

      bt0 = 4 * wid + 2 * h
      for btl in range(2):
        pltpu.sync_copy(o.at[btl, :, :, pl.ds(0, BI)],
                        out_hbm.at[l, :, bt0 + btl])

    pltpu.sync_copy(xt_hbm.at[:, pl.ds(BPW * wid, BPW)], idx_all)
    fire(0, 0, 0)

    @pl.loop(0, L)
    def _(l):
      fire(l, 1, 1)
      drain(0)
      fill_store(l, 0, 0)

      @pl.when(l + 1 < L)
      def _():
        fire(l + 1, 0, 0)

      drain(1)
      fill_store(l, 1, 1)

  return embed


@jax.jit
def kernel(x, table):
  xt = jnp.swapaxes(x.astype(jnp.int32), 0, 1)
  out5 = _build()(xt, table)
  return out5.transpose(2, 4, 0, 1, 3).reshape(B, L, D)


# speedup vs baseline: 1.7984x; 1.0261x over previous
import functools
import math

import jax
import jax.numpy as jnp
from jax import lax
from jax.experimental import pallas as pl
from jax.experimental.pallas import tpu as pltpu
from jax.experimental.pallas import tpu_sc as plsc

B = 16384
L = 50
D = 64
LANES = 16
NC, NS = 2, 16
NW = NC * NS
BPW = B // NW
HALF = BPW // 2
DT, DI = D // 8, 8
BT, BI = B // 128, 128
SCALE = math.sqrt(D)


def _build():
  mesh = plsc.VectorSubcoreMesh(core_axis_name="c", subcore_axis_name="s")

  @functools.partial(
      pl.kernel,
      out_type=jax.ShapeDtypeStruct((L, DT, BT, DI, BI), jnp.float32),
      mesh=mesh,
      compiler_params=pltpu.CompilerParams(
          use_tc_tiling_on_sc=False, needs_layout_passes=False),
      scratch_types=[
          pltpu.VMEM((L, BPW), jnp.int32),
          pltpu.VMEM((HALF, D), jnp.float32),
          pltpu.VMEM((HALF, D), jnp.float32),
          pltpu.VMEM((2, DT, DI, BI + 1), jnp.float32),
          pltpu.VMEM((2, DT, DI, BI + 1), jnp.float32),
          pltpu.SemaphoreType.DMA,
          pltpu.SemaphoreType.DMA,
          pltpu.SemaphoreType.DMA,
          pltpu.SemaphoreType.DMA,
      ],
  )
  def embed(xt_hbm, table_hbm, out_hbm, idx_all, rows0, rows1,
            tile0, tile1, gsem0, gsem1, ssem0, ssem1):
    wid = lax.axis_index("s") * NC + lax.axis_index("c")
    rows_b = (rows0, rows1)
    tile_b = (tile0, tile1)
    gsem = (gsem0, gsem1)
    ssem = (ssem0, ssem1)
    iota = lax.iota(jnp.int32, LANES)

    def fire(l, h, bufi):
      for j in range(2):
        idx_sl = idx_all.at[l, pl.ds(HALF * h + 128 * j, 128)]
        pltpu.async_copy(table_hbm.at[idx_sl],
                         rows_b[bufi].at[pl.ds(128 * j, 128)], gsem[bufi])

    def drain(bufi):
      for j in range(2):
        idx_sl = idx_all.at[0, pl.ds(128 * j, 128)]
        pltpu.make_async_copy(table_hbm.at[idx_sl],
                              rows_b[bufi].at[pl.ds(128 * j, 128)],
                              gsem[bufi]).wait()

    dtv = [(16 * c + iota) >> 3 for c in range(D // LANES)]
    div = iota & 7

    def fill_store(l, h, bufi):
      r = rows_b[bufi]
      o = tile_b[bufi]

      @pl.when(l > 0)
      def _():
        for btl in range(2):
          pltpu.make_async_copy(o.at[btl, :, :, pl.ds(0, BI)],
                                out_hbm.at[0, :, btl], ssem[bufi]).wait()

      for btl in range(2):
        bv = jnp.full((LANES,), btl, jnp.int32)

        @pl.loop(0, 128, unroll=4)
        def _(rr):
          row = 128 * btl + rr
          biv = jnp.full((LANES,), rr, jnp.int32)
          for c in range(D // LANES):
            v = r[row, pl.ds(16 * c, 16)] * SCALE
            plsc.store_scatter(o, [bv, dtv[c], div, biv], v)

      bt0 = 4 * wid + 2 * h
      for btl in range(2):
        pltpu.async_copy(o.at[btl, :, :, pl.ds(0, BI)],
                         out_hbm.at[l, :, bt0 + btl], ssem[bufi])

    pltpu.sync_copy(xt_hbm.at[:, pl.ds(BPW * wid, BPW)], idx_all)
    fire(0, 0, 0)

    @pl.loop(0, L)
    def _(l):
      fire(l, 1, 1)
      drain(0)
      fill_store(l, 0, 0)

      @pl.when(l + 1 < L)
      def _():
        fire(l + 1, 0, 0)

      drain(1)
      fill_store(l, 1, 1)

    for bufi in range(2):
      for btl in range(2):
        pltpu.make_async_copy(tile_b[bufi].at[btl, :, :, pl.ds(0, BI)],
                              out_hbm.at[0, :, btl], ssem[bufi]).wait()

  return embed


@jax.jit
def kernel(x, table):
  xt = jnp.swapaxes(x.astype(jnp.int32), 0, 1)
  out5 = _build()(xt, table)
  return out5.transpose(2, 4, 0, 1, 3).reshape(B, L, D)
